# baseline (speedup 1.0000x reference)
import functools

import jax
import jax.numpy as jnp
from jax import lax
from jax.experimental import pallas as pl
from jax.experimental.pallas import tpu as pltpu
from jax.experimental.pallas import tpu_sc as plsc

_N = 10000
_D = 128
_E = 320000
_NPAD = 10240
_CHUNK = 64
_NC, _NS = 2, 16
_NW = _NC * _NS
_EPAD = 327680
_NCHUNKS = _EPAD // _CHUNK
_CPW = _NCHUNKS // _NW
_EPW = _EPAD // _NW
_NBUF = 4
_NPH = 2
_CPP = _CPW // _NPH
_EPP = _CPP * _CHUNK
_RPS = _NPAD // _NS

_mesh = plsc.VectorSubcoreMesh(
    core_axis_name="c", subcore_axis_name="s", num_cores=_NC, num_subcores=_NS
)


@functools.partial(
    pl.kernel,
    out_type=jax.ShapeDtypeStruct((_NW, _NPAD), jnp.float32),
    mesh=_mesh,
    scratch_types=[
        pltpu.VMEM((_EPW,), jnp.int32),
        pltpu.VMEM((_NPAD,), jnp.float32),
    ],
    compiler_params=pltpu.CompilerParams(needs_layout_passes=False),
)
def _deg_kernel(dst_hbm, out_hbm, dstv, hist):
    c = lax.axis_index("c")
    s = lax.axis_index("s")
    w = s * _NC + c
    pltpu.sync_copy(dst_hbm.at[pl.ds(w * _EPW, _EPW)], dstv)

    zeros = jnp.zeros((16,), jnp.float32)

    def zbody(i, carry):
        hist[pl.ds(i * 16, 16)] = zeros
        return carry

    lax.fori_loop(0, _NPAD // 16, zbody, 0)

    ones = jnp.ones((16,), jnp.float32)

    def hbody(i, carry):
        idx = dstv[pl.ds(i * 16, 16)]
        plsc.addupdate_scatter(hist, [idx], ones)
        return carry

    lax.fori_loop(0, _EPW // 16, hbody, 0)
    pltpu.sync_copy(hist, out_hbm.at[w])


@functools.partial(
    pl.kernel,
    out_type=jax.ShapeDtypeStruct((_NC, _NPAD, _D), jnp.float32),
    mesh=_mesh,
    scratch_types=(
        [
            pltpu.VMEM((_EPP,), jnp.int32),
            pltpu.VMEM((_CPP, _CHUNK), jnp.int32),
            pltpu.VMEM_SHARED((_NPAD, _D), jnp.float32),
        ]
        + [pltpu.VMEM((_CHUNK, _D), jnp.float32)] * _NBUF
        + [pltpu.SemaphoreType.DMA] * (2 * _NBUF + 1)
    ),
    compiler_params=pltpu.CompilerParams(needs_layout_passes=False),
)
def _scatter_kernel(g_hbm, src_hbm, dst3_hbm, out_hbm,
                    sidx_all, didx_all, acc, *bufs_and_sems):
    rows = bufs_and_sems[:_NBUF]
    gsems = bufs_and_sems[_NBUF:2 * _NBUF]
    ssems = bufs_and_sems[2 * _NBUF:3 * _NBUF]
    isem = bufs_and_sems[3 * _NBUF]
    c = lax.axis_index("c")
    s = lax.axis_index("s")
    w = s * _NC + c

    icp1 = pltpu.async_copy(
        src_hbm.at[pl.ds(w * _EPW, _EPP)], sidx_all, isem)
    icp2 = pltpu.async_copy(dst3_hbm.at[w, pl.ds(0, _CPP)], didx_all, isem)

    zeros = jnp.zeros((16,), jnp.float32)

    def zbody(i, carry):
        rows[0][i // 8, pl.ds((i % 8) * 16, 16)] = zeros
        return carry

    lax.fori_loop(0, _CHUNK * 8, zbody, 0)

    def zcbody(t, carry):
        pltpu.sync_copy(rows[0],
                        acc.at[pl.ds(s * _RPS + t * _CHUNK, _CHUNK)])
        return carry

    lax.fori_loop(0, _RPS // _CHUNK, zcbody, 0)
    icp1.wait()
    icp2.wait()

    def start_gather(i, k):
        pltpu.async_copy(
            g_hbm.at[sidx_all.at[pl.ds(i * _CHUNK, _CHUNK)]],
            rows[k], gsems[k])

    def start_scatter(i, k):
        pltpu.async_copy(rows[k], acc.at[didx_all.at[i]], ssems[k], add=True)

    def wait_gather(k):
        pltpu.make_async_copy(
            g_hbm.at[sidx_all.at[pl.ds(0, _CHUNK)]], rows[k], gsems[k]).wait()

    def wait_scatter(k):
        pltpu.make_async_copy(
            rows[k], acc.at[didx_all.at[0]], ssems[k]).wait()

    plsc.subcore_barrier()

    rounds = _CPP // _NBUF

    def body(j, carry):
        for k in range(_NBUF):
            wait_gather(k)
            start_scatter((j - 1) * _NBUF + k, k)
        for k in range(_NBUF):
            wait_scatter(k)
            start_gather(j * _NBUF + k, k)
        return carry

    for ph in range(_NPH):
        if ph > 0:
            pltpu.sync_copy(
                src_hbm.at[pl.ds(w * _EPW + ph * _EPP, _EPP)], sidx_all)
            pltpu.sync_copy(dst3_hbm.at[w, pl.ds(ph * _CPP, _CPP)], didx_all)
        for k in range(_NBUF):
            start_gather(k, k)
        lax.fori_loop(1, rounds, body, 0)
        for k in range(_NBUF):
            wait_gather(k)
            start_scatter((rounds - 1) * _NBUF + k, k)
        for k in range(_NBUF):
            wait_scatter(k)
    plsc.subcore_barrier()
    pltpu.sync_copy(acc.at[pl.ds(s * _RPS, _RPS)],
                    out_hbm.at[c, pl.ds(s * _RPS, _RPS)])


_BR = 2048
_GRID = _NPAD // _BR


def _dense1a_body(x_ref, w1_ref, h_ref):
    h_ref[...] = jnp.dot(x_ref[...], w1_ref[...],
                         preferred_element_type=jnp.float32)


def _dense1b_body(h_ref, degp_ref, g1_ref, dinv_ref):
    deg = jnp.sum(degp_ref[...], axis=0) + 1.0
    dinv = lax.rsqrt(deg)
    g1_ref[...] = h_ref[...] * dinv[:, None]
    dinv_ref[...] = dinv[:, None]


def _dense2_body(sp_ref, g1_ref, dinv_ref, b1_ref, w2_ref, g2_ref):
    ssum = sp_ref[0] + sp_ref[1] + g1_ref[...]
    y = jnp.maximum(ssum * dinv_ref[...] + b1_ref[...], 0.0)
    g2_ref[...] = (
        jnp.dot(y, w2_ref[...], preferred_element_type=jnp.float32)
        * dinv_ref[...]
    )


def _dense3_body(sp_ref, g2_ref, dinv_ref, b2_ref, z_ref):
    z_ref[...] = (sp_ref[0] + sp_ref[1] + g2_ref[...]) * dinv_ref[...] + b2_ref[...]


_dense1a = pl.pallas_call(
    _dense1a_body,
    grid=(_GRID,),
    in_specs=[
        pl.BlockSpec((_BR, _D), lambda i: (i, 0)),
        pl.BlockSpec((_D, _D), lambda i: (0, 0)),
    ],
    out_specs=pl.BlockSpec((_BR, _D), lambda i: (i, 0)),
    out_shape=jax.ShapeDtypeStruct((_N, _D), jnp.float32),
)

_dense1b = pl.pallas_call(
    _dense1b_body,
    grid=(_GRID,),
    in_specs=[
        pl.BlockSpec((_BR, _D), lambda i: (i, 0)),
        pl.BlockSpec((_NW, _BR), lambda i: (0, i)),
    ],
    out_specs=[
        pl.BlockSpec((_BR, _D), lambda i: (i, 0)),
        pl.BlockSpec((_BR, 1), lambda i: (i, 0)),
    ],
    out_shape=[
        jax.ShapeDtypeStruct((_N, _D), jnp.float32),
        jax.ShapeDtypeStruct((_N, 1), jnp.float32),
    ],
)

_dense2 = pl.pallas_call(
    _dense2_body,
    grid=(_GRID,),
    in_specs=[
        pl.BlockSpec((_NC, _BR, _D), lambda i: (0, i, 0)),
        pl.BlockSpec((_BR, _D), lambda i: (i, 0)),
        pl.BlockSpec((_BR, 1), lambda i: (i, 0)),
        pl.BlockSpec((1, _D), lambda i: (0, 0)),
        pl.BlockSpec((_D, _D), lambda i: (0, 0)),
    ],
    out_specs=pl.BlockSpec((_BR, _D), lambda i: (i, 0)),
    out_shape=jax.ShapeDtypeStruct((_N, _D), jnp.float32),
)

_dense3 = pl.pallas_call(
    _dense3_body,
    grid=(_GRID,),
    in_specs=[
        pl.BlockSpec((_NC, _BR, _D), lambda i: (0, i, 0)),
        pl.BlockSpec((_BR, _D), lambda i: (i, 0)),
        pl.BlockSpec((_BR, 1), lambda i: (i, 0)),
        pl.BlockSpec((1, _D), lambda i: (0, 0)),
    ],
    out_specs=pl.BlockSpec((_BR, _D), lambda i: (i, 0)),
    out_shape=jax.ShapeDtypeStruct((_N, _D), jnp.float32),
)


@jax.jit
def kernel(x, edge_index, W1, b1, W2, b2):
    src = edge_index[0]
    dst = edge_index[1]
    pad = _EPAD - _E
    ar = jnp.arange(pad, dtype=jnp.int32)
    src_p = jnp.concatenate([src, ar % _N])
    dst_p = jnp.concatenate([dst, _N + ar % (_NPAD - _N)])
    dst3 = dst_p.reshape(_NW, _CPW, _CHUNK)

    degp = _deg_kernel(dst_p)

    h1 = _dense1a(x, W1)
    g1, dinv = _dense1b(h1, degp)
    s1 = _scatter_kernel(g1, src_p, dst3)
    g2 = _dense2(s1, g1, dinv, b1.reshape(1, _D), W2)
    s2 = _scatter_kernel(g2, src_p, dst3)
    return _dense3(s2, g2, dinv, b2.reshape(1, _D))


# speedup vs baseline: 1.0023x; 1.0023x over previous
import functools

import jax
import jax.numpy as jnp
from jax import lax
from jax.experimental import pallas as pl
from jax.experimental.pallas import tpu as pltpu
from jax.experimental.pallas import tpu_sc as plsc

_N = 10000
_D = 128
_E = 320000
_NPAD = 10240
_CHUNK = 64
_NC, _NS = 2, 16
_NW = _NC * _NS
_EPAD = 327680
_NCHUNKS = _EPAD // _CHUNK
_CPW = _NCHUNKS // _NW
_EPW = _EPAD // _NW
_NBUF = 4
_NPH = 2
_CPP = _CPW // _NPH
_EPP = _CPP * _CHUNK
_RPS = _NPAD // _NS

_mesh = plsc.VectorSubcoreMesh(
    core_axis_name="c", subcore_axis_name="s", num_cores=_NC, num_subcores=_NS
)


@functools.partial(
    pl.kernel,
    out_type=jax.ShapeDtypeStruct((_NW, _NPAD), jnp.float32),
    mesh=_mesh,
    scratch_types=[
        pltpu.VMEM((_EPW,), jnp.int32),
        pltpu.VMEM((_NPAD,), jnp.float32),
    ],
    compiler_params=pltpu.CompilerParams(needs_layout_passes=False),
)
def _deg_kernel(dst_hbm, out_hbm, dstv, hist):
    c = lax.axis_index("c")
    s = lax.axis_index("s")
    w = s * _NC + c
    pltpu.sync_copy(dst_hbm.at[pl.ds(w * _EPW, _EPW)], dstv)

    zeros = jnp.zeros((16,), jnp.float32)

    def zbody(i, carry):
        hist[pl.ds(i * 16, 16)] = zeros
        return carry

    lax.fori_loop(0, _NPAD // 16, zbody, 0)

    ones = jnp.ones((16,), jnp.float32)

    def hbody(i, carry):
        for u in range(4):
            idx = dstv[pl.ds((i * 4 + u) * 16, 16)]
            plsc.addupdate_scatter(hist, [idx], ones)
        return carry

    lax.fori_loop(0, _EPW // 64, hbody, 0)
    pltpu.sync_copy(hist, out_hbm.at[w])


@functools.partial(
    pl.kernel,
    out_type=jax.ShapeDtypeStruct((_NC, _NPAD, _D), jnp.float32),
    mesh=_mesh,
    scratch_types=(
        [
            pltpu.VMEM((_EPP,), jnp.int32),
            pltpu.VMEM((_CPP, _CHUNK), jnp.int32),
            pltpu.VMEM_SHARED((_NPAD, _D), jnp.float32),
        ]
        + [pltpu.VMEM((_CHUNK, _D), jnp.float32)] * _NBUF
        + [pltpu.SemaphoreType.DMA] * (2 * _NBUF + 1)
    ),
    compiler_params=pltpu.CompilerParams(needs_layout_passes=False),
)
def _scatter_kernel(g_hbm, src_hbm, dst3_hbm, out_hbm,
                    sidx_all, didx_all, acc, *bufs_and_sems):
    rows = bufs_and_sems[:_NBUF]
    gsems = bufs_and_sems[_NBUF:2 * _NBUF]
    ssems = bufs_and_sems[2 * _NBUF:3 * _NBUF]
    isem = bufs_and_sems[3 * _NBUF]
    c = lax.axis_index("c")
    s = lax.axis_index("s")
    w = s * _NC + c

    icp1 = pltpu.async_copy(
        src_hbm.at[pl.ds(w * _EPW, _EPP)], sidx_all, isem)
    icp2 = pltpu.async_copy(dst3_hbm.at[w, pl.ds(0, _CPP)], didx_all, isem)

    zeros = jnp.zeros((16,), jnp.float32)

    def zbody(i, carry):
        rows[0][i // 8, pl.ds((i % 8) * 16, 16)] = zeros
        return carry

    lax.fori_loop(0, _CHUNK * 8, zbody, 0)

    def zcbody(t, carry):
        pltpu.sync_copy(rows[0],
                        acc.at[pl.ds(s * _RPS + t * _CHUNK, _CHUNK)])
        return carry

    lax.fori_loop(0, _RPS // _CHUNK, zcbody, 0)
    icp1.wait()
    icp2.wait()

    def start_gather(i, k):
        pltpu.async_copy(
            g_hbm.at[sidx_all.at[pl.ds(i * _CHUNK, _CHUNK)]],
            rows[k], gsems[k])

    def start_scatter(i, k):
        pltpu.async_copy(rows[k], acc.at[didx_all.at[i]], ssems[k], add=True)

    def wait_gather(k):
        pltpu.make_async_copy(
            g_hbm.at[sidx_all.at[pl.ds(0, _CHUNK)]], rows[k], gsems[k]).wait()

    def wait_scatter(k):
        pltpu.make_async_copy(
            rows[k], acc.at[didx_all.at[0]], ssems[k]).wait()

    plsc.subcore_barrier()

    rounds = _CPP // _NBUF

    def body(j, carry):
        for k in range(_NBUF):
            wait_gather(k)
            start_scatter((j - 1) * _NBUF + k, k)
        for k in range(_NBUF):
            wait_scatter(k)
            start_gather(j * _NBUF + k, k)
        return carry

    for ph in range(_NPH):
        if ph > 0:
            pltpu.sync_copy(
                src_hbm.at[pl.ds(w * _EPW + ph * _EPP, _EPP)], sidx_all)
            pltpu.sync_copy(dst3_hbm.at[w, pl.ds(ph * _CPP, _CPP)], didx_all)
        for k in range(_NBUF):
            start_gather(k, k)
        lax.fori_loop(1, rounds, body, 0)
        for k in range(_NBUF):
            wait_gather(k)
            start_scatter((rounds - 1) * _NBUF + k, k)
        for k in range(_NBUF):
            wait_scatter(k)
    plsc.subcore_barrier()
    pltpu.sync_copy(acc.at[pl.ds(s * _RPS, _RPS)],
                    out_hbm.at[c, pl.ds(s * _RPS, _RPS)])


_BR = 2048
_GRID = _NPAD // _BR


def _dense1_body(x_ref, w1_ref, degp_ref, g1_ref, dinv_ref):
    deg = jnp.sum(degp_ref[...], axis=0) + 1.0
    dinv = lax.rsqrt(deg)
    h = jnp.dot(x_ref[...], w1_ref[...], preferred_element_type=jnp.float32)
    g1_ref[...] = h * dinv[:, None]
    dinv_ref[...] = dinv[:, None]


def _dense2_body(sp_ref, g1_ref, dinv_ref, b1_ref, w2_ref, g2_ref):
    ssum = sp_ref[0] + sp_ref[1] + g1_ref[...]
    y = jnp.maximum(ssum * dinv_ref[...] + b1_ref[...], 0.0)
    g2_ref[...] = (
        jnp.dot(y, w2_ref[...], preferred_element_type=jnp.float32)
        * dinv_ref[...]
    )


def _dense3_body(sp_ref, g2_ref, dinv_ref, b2_ref, z_ref):
    z_ref[...] = (sp_ref[0] + sp_ref[1] + g2_ref[...]) * dinv_ref[...] + b2_ref[...]


_dense1 = pl.pallas_call(
    _dense1_body,
    grid=(_GRID,),
    in_specs=[
        pl.BlockSpec((_BR, _D), lambda i: (i, 0)),
        pl.BlockSpec((_D, _D), lambda i: (0, 0)),
        pl.BlockSpec((_NW, _BR), lambda i: (0, i)),
    ],
    out_specs=[
        pl.BlockSpec((_BR, _D), lambda i: (i, 0)),
        pl.BlockSpec((_BR, 1), lambda i: (i, 0)),
    ],
    out_shape=[
        jax.ShapeDtypeStruct((_N, _D), jnp.float32),
        jax.ShapeDtypeStruct((_N, 1), jnp.float32),
    ],
)

_dense2 = pl.pallas_call(
    _dense2_body,
    grid=(_GRID,),
    in_specs=[
        pl.BlockSpec((_NC, _BR, _D), lambda i: (0, i, 0)),
        pl.BlockSpec((_BR, _D), lambda i: (i, 0)),
        pl.BlockSpec((_BR, 1), lambda i: (i, 0)),
        pl.BlockSpec((1, _D), lambda i: (0, 0)),
        pl.BlockSpec((_D, _D), lambda i: (0, 0)),
    ],
    out_specs=pl.BlockSpec((_BR, _D), lambda i: (i, 0)),
    out_shape=jax.ShapeDtypeStruct((_N, _D), jnp.float32),
)

_dense3 = pl.pallas_call(
    _dense3_body,
    grid=(_GRID,),
    in_specs=[
        pl.BlockSpec((_NC, _BR, _D), lambda i: (0, i, 0)),
        pl.BlockSpec((_BR, _D), lambda i: (i, 0)),
        pl.BlockSpec((_BR, 1), lambda i: (i, 0)),
        pl.BlockSpec((1, _D), lambda i: (0, 0)),
    ],
    out_specs=pl.BlockSpec((_BR, _D), lambda i: (i, 0)),
    out_shape=jax.ShapeDtypeStruct((_N, _D), jnp.float32),
)


@jax.jit
def kernel(x, edge_index, W1, b1, W2, b2):
    src = edge_index[0]
    dst = edge_index[1]
    pad = _EPAD - _E
    ar = jnp.arange(pad, dtype=jnp.int32)
    src_p = jnp.concatenate([src, ar % _N])
    dst_p = jnp.concatenate([dst, _N + ar % (_NPAD - _N)])
    dst3 = dst_p.reshape(_NW, _CPW, _CHUNK)

    degp = _deg_kernel(dst_p)

    g1, dinv = _dense1(x, W1, degp)
    s1 = _scatter_kernel(g1, src_p, dst3)
    g2 = _dense2(s1, g1, dinv, b1.reshape(1, _D), W2)
    s2 = _scatter_kernel(g2, src_p, dst3)
    return _dense3(s2, g2, dinv, b2.reshape(1, _D))
